# baseline (speedup 1.0000x reference)
import jax
import jax.numpy as jnp
from jax.experimental import pallas as pl

_TOKENS = 16384
_PATHS = 64
_BLOCK_ROWS = 8192


def _body(s_ref, o_ref):
    s = s_ref[...]
    pos = s > 0.0
    col = jax.lax.broadcasted_iota(jnp.int32, s.shape, 1)
    rmax = jnp.max(s, axis=1, keepdims=True)
    resid = (col == 0) & (rmax <= 0.0)
    o_ref[...] = jnp.where(pos | resid, 1, 0).astype(jnp.int32)


def kernel(score):
    return pl.pallas_call(
        _body,
        out_shape=jax.ShapeDtypeStruct((_TOKENS, _PATHS), jnp.int32),
        grid=(_TOKENS // _BLOCK_ROWS,),
        in_specs=[pl.BlockSpec((_BLOCK_ROWS, _PATHS), lambda i: (i, 0))],
        out_specs=pl.BlockSpec((_BLOCK_ROWS, _PATHS), lambda i: (i, 0)),
    )(score)
